# Initial kernel scaffold; baseline (speedup 1.0000x reference)
import jax
import jax.numpy as jnp
from jax.experimental import pallas as pl


def kernel(x, edge_index, batch, W1, att_src, att_dst, b1, W2, b2, Wfc, bfc):
    raise NotImplementedError("write your pallas kernel here")


# speedup vs baseline: 9.7961x; 9.7961x over previous
import jax
import jax.numpy as jnp
from jax.experimental import pallas as pl
from jax.experimental.pallas import tpu as pltpu
from jax.experimental.pallas import tpu_sc as plsc

N = 10000
D = 128
H = 64
G = 64
N_PAD = 10240
E = 320000
NT = 32
C = 128
CPT = 79
EPT = C * CPT
E_PAD = NT * EPT
BLK = 256
NBLK = N_PAD // BLK
RPS = N_PAD // 16

_HI = jax.lax.Precision.HIGHEST


def _tc1_body(x_ref, w1_ref, asrc_ref, adst_ref, h0_ref, h1_ref, as_ref, ad_ref):
    h = jnp.dot(x_ref[...], w1_ref[...], precision=_HI)
    h0_ref[...] = h[:, :H]
    h1_ref[...] = h[:, H:]
    as_ref[...] = jnp.dot(h, asrc_ref[...], precision=_HI)
    ad_ref[...] = jnp.dot(h, adst_ref[...], precision=_HI)


def _tc1_call(x_pad, W1, att_src, att_dst):
    return pl.pallas_call(
        _tc1_body,
        grid=(NBLK,),
        in_specs=[
            pl.BlockSpec((BLK, D), lambda i: (i, 0)),
            pl.BlockSpec((D, D), lambda i: (0, 0)),
            pl.BlockSpec((D,), lambda i: (0,)),
            pl.BlockSpec((D,), lambda i: (0,)),
        ],
        out_specs=[
            pl.BlockSpec((BLK, H), lambda i: (i, 0)),
            pl.BlockSpec((BLK, H), lambda i: (i, 0)),
            pl.BlockSpec((BLK,), lambda i: (i,)),
            pl.BlockSpec((BLK,), lambda i: (i,)),
        ],
        out_shape=[
            jax.ShapeDtypeStruct((N_PAD, H), jnp.float32),
            jax.ShapeDtypeStruct((N_PAD, H), jnp.float32),
            jax.ShapeDtypeStruct((N_PAD,), jnp.float32),
            jax.ShapeDtypeStruct((N_PAD,), jnp.float32),
        ],
    )(x_pad, W1, att_src, att_dst)


def _tc2_body(acc_ref, scal_ref, b1_ref, w2_ref, h2a_ref, h2b_ref, dinv_ref):
    a = jnp.concatenate(
        [acc_ref[0, 0] + acc_ref[1, 0], acc_ref[0, 1] + acc_ref[1, 1]], axis=-1
    )
    den = scal_ref[0, :, 0] + scal_ref[1, :, 0]
    deg = scal_ref[0, :, 1] + scal_ref[1, :, 1] + 1.0
    x2 = jnp.maximum(a / (den + 1e-16)[:, None] + b1_ref[...], 0.0)
    h2 = jnp.dot(x2, w2_ref[...], precision=_HI)
    h2a_ref[...] = h2[:, :H]
    h2b_ref[...] = h2[:, H:]
    dinv_ref[...] = 1.0 / jnp.sqrt(deg)


def _tc2_call(acc1, scal, b1, W2):
    return pl.pallas_call(
        _tc2_body,
        grid=(NBLK,),
        in_specs=[
            pl.BlockSpec((2, 2, BLK, H), lambda i: (0, 0, i, 0)),
            pl.BlockSpec((2, BLK, 16), lambda i: (0, i, 0)),
            pl.BlockSpec((D,), lambda i: (0,)),
            pl.BlockSpec((D, D), lambda i: (0, 0)),
        ],
        out_specs=[
            pl.BlockSpec((BLK, H), lambda i: (i, 0)),
            pl.BlockSpec((BLK, H), lambda i: (i, 0)),
            pl.BlockSpec((BLK,), lambda i: (i,)),
        ],
        out_shape=[
            jax.ShapeDtypeStruct((N_PAD, H), jnp.float32),
            jax.ShapeDtypeStruct((N_PAD, H), jnp.float32),
            jax.ShapeDtypeStruct((N_PAD,), jnp.float32),
        ],
    )(acc1, scal, b1, W2)


def _tc3a_body(acc2_ref, h2a_ref, h2b_ref, dinv_ref, b2_ref, batch_ref,
               sums_ref, cnts_ref):
    i = pl.program_id(0)
    dinv = dinv_ref[...]
    h2 = jnp.concatenate([h2a_ref[...], h2b_ref[...]], axis=-1)
    a2 = jnp.concatenate(
        [acc2_ref[0, 0] + acc2_ref[1, 0], acc2_ref[0, 1] + acc2_ref[1, 1]], axis=-1
    )
    x3 = jnp.maximum(a2 + (dinv * dinv)[:, None] * h2 + b2_ref[...], 0.0)
    bt = batch_ref[0, 0, :]
    oh = (bt[:, None] == jax.lax.broadcasted_iota(jnp.int32, (BLK, G), 1)).astype(jnp.float32)
    ps = jax.lax.dot_general(oh, x3, (((0,), (0,)), ((), ())), precision=_HI)
    cnt = jnp.sum(oh, axis=0)

    @pl.when(i == 0)
    def _():
        sums_ref[...] = jnp.zeros_like(sums_ref)
        cnts_ref[...] = jnp.zeros_like(cnts_ref)

    sums_ref[...] += ps
    cnts_ref[...] += jnp.broadcast_to(cnt[:, None], (G, D))


def _tc3a_call(acc2, h2a, h2b, dinv, b2, batch3):
    return pl.pallas_call(
        _tc3a_body,
        grid=(NBLK,),
        in_specs=[
            pl.BlockSpec((2, 2, BLK, H), lambda i: (0, 0, i, 0)),
            pl.BlockSpec((BLK, H), lambda i: (i, 0)),
            pl.BlockSpec((BLK, H), lambda i: (i, 0)),
            pl.BlockSpec((BLK,), lambda i: (i,)),
            pl.BlockSpec((D,), lambda i: (0,)),
            pl.BlockSpec((1, 1, BLK), lambda i: (i, 0, 0)),
        ],
        out_specs=[
            pl.BlockSpec((G, D), lambda i: (0, 0)),
            pl.BlockSpec((G, D), lambda i: (0, 0)),
        ],
        out_shape=[
            jax.ShapeDtypeStruct((G, D), jnp.float32),
            jax.ShapeDtypeStruct((G, D), jnp.float32),
        ],
    )(acc2, h2a, h2b, dinv, b2, batch3)


def _tc3b_body(sums_ref, cnts_ref, wfc_ref, bfc_ref, out_ref):
    g = sums_ref[...] / jnp.maximum(cnts_ref[...], 1.0)
    out_ref[...] = jnp.dot(g, wfc_ref[...], precision=_HI) + bfc_ref[...]


def _tc3b_call(sums, cnts, Wfc, bfc):
    return pl.pallas_call(
        _tc3b_body,
        out_shape=jax.ShapeDtypeStruct((G, Wfc.shape[1]), jnp.float32),
    )(sums, cnts, Wfc, bfc)


def _sc_mesh():
    return plsc.VectorSubcoreMesh(core_axis_name="c", subcore_axis_name="s")


_SC_PARAMS = pltpu.CompilerParams(
    needs_layout_passes=False, use_tc_tiling_on_sc=False
)


def _zero_rows(ref, nrows, width):
    zf = jnp.zeros((16,), jnp.float32)

    @pl.loop(0, nrows)
    def _(r):
        for k in range(width // 16):
            ref[r, pl.ds(k * 16, 16)] = zf


def _zero_acc(zrows, acc_sh, row0):
    @pl.loop(0, RPS // C)
    def _(b):
        pltpu.sync_copy(zrows, acc_sh.at[pl.ds(row0 + b * C, C)])


def _sc_gat_body(h0_hbm, h1_hbm, src_hbm, dst_hbm, as_hbm, ad_hbm,
                 acc_out, scal_out,
                 asv, adv, sidx, didx, rows, zrows, pbuf, scalbuf, z16,
                 acc_sh, scal_sh):
    cid = jax.lax.axis_index("c")
    sid = jax.lax.axis_index("s")
    wid = sid * 2 + cid
    pltpu.sync_copy(as_hbm, asv)
    pltpu.sync_copy(ad_hbm, adv)
    zf = jnp.zeros((16,), jnp.float32)
    lane1 = (jnp.arange(16, dtype=jnp.int32) == 1).astype(jnp.float32)

    _zero_rows(zrows, C, H)

    @pl.loop(0, 64)
    def _(r):
        z16[r] = zf

    @pl.loop(0, C)
    def _(r):
        scalbuf[r] = lane1

    row0 = sid * RPS
    _zero_acc(zrows, acc_sh, row0)

    @pl.loop(0, 10)
    def _(b):
        pltpu.sync_copy(z16, scal_sh.at[pl.ds(row0 + b * 64, 64)])

    plsc.subcore_barrier()

    iota16 = jnp.arange(16, dtype=jnp.int32)
    zi = jnp.zeros((16,), jnp.int32)
    ebase = wid * EPT

    def edge_pass(h_hbm, do_scal):
        @pl.loop(0, CPT)
        def _(c):
            off = ebase + c * C
            pltpu.sync_copy(src_hbm.at[pl.ds(off, C)], sidx)
            pltpu.sync_copy(dst_hbm.at[pl.ds(off, C)], didx)

            @pl.loop(0, 8)
            def _(g):
                s16 = sidx[pl.ds(g * 16, 16)]
                d16 = didx[pl.ds(g * 16, 16)]
                e16 = plsc.load_gather(asv, [s16]) + plsc.load_gather(adv, [d16])
                e16 = jnp.where(e16 < 0, e16 * 0.2, e16)
                p16 = jnp.exp(e16)
                pbuf[pl.ds(g * 16, 16)] = p16
                if do_scal:
                    plsc.store_scatter(scalbuf, [g * 16 + iota16, zi], p16)

            pltpu.sync_copy(h_hbm.at[sidx], rows)

            @pl.loop(0, 8)
            def _(g):
                for j in range(16):
                    pj = plsc.load_gather(pbuf, [zi + (g * 16 + j)])
                    for r in range(H // 16):
                        rows[g * 16 + j, pl.ds(r * 16, 16)] = (
                            rows[g * 16 + j, pl.ds(r * 16, 16)] * pj
                        )

            pltpu.sync_copy(rows, acc_sh.at[didx], add=True)
            if do_scal:
                pltpu.sync_copy(scalbuf, scal_sh.at[didx], add=True)

    def write_acc(half):
        @pl.loop(0, RPS // C)
        def _(b):
            r0 = row0 + b * C
            pltpu.sync_copy(acc_sh.at[pl.ds(r0, C)],
                            acc_out.at[cid, half, pl.ds(r0, C)])

    edge_pass(h0_hbm, do_scal=True)
    plsc.subcore_barrier()
    write_acc(0)

    @pl.loop(0, 10)
    def _(b):
        r0 = row0 + b * 64
        pltpu.sync_copy(scal_sh.at[pl.ds(r0, 64)], scal_out.at[cid, pl.ds(r0, 64)])

    plsc.subcore_barrier()
    _zero_acc(zrows, acc_sh, row0)
    plsc.subcore_barrier()
    edge_pass(h1_hbm, do_scal=False)
    plsc.subcore_barrier()
    write_acc(1)


def _sc_gat_call(h0, h1, src_p, dst_p, a_src, a_dst):
    kfn = pl.kernel(
        _sc_gat_body,
        out_type=[
            jax.ShapeDtypeStruct((2, 2, N_PAD, H), jnp.float32),
            jax.ShapeDtypeStruct((2, N_PAD, 16), jnp.float32),
        ],
        mesh=_sc_mesh(),
        scratch_types=[
            pltpu.VMEM((N_PAD,), jnp.float32),
            pltpu.VMEM((N_PAD,), jnp.float32),
            pltpu.VMEM((C,), jnp.int32),
            pltpu.VMEM((C,), jnp.int32),
            pltpu.VMEM((C, H), jnp.float32),
            pltpu.VMEM((C, H), jnp.float32),
            pltpu.VMEM((C,), jnp.float32),
            pltpu.VMEM((C, 16), jnp.float32),
            pltpu.VMEM((64, 16), jnp.float32),
            pltpu.VMEM_SHARED((N_PAD, H), jnp.float32),
            pltpu.VMEM_SHARED((N_PAD, 16), jnp.float32),
        ],
        compiler_params=_SC_PARAMS,
    )
    return kfn(h0, h1, src_p, dst_p, a_src, a_dst)


def _sc_gcn_body(h2a_hbm, h2b_hbm, src_hbm, dst_hbm, dinv_hbm, acc_out,
                 dv, sidx, didx, rows, zrows, nbuf, acc_sh):
    cid = jax.lax.axis_index("c")
    sid = jax.lax.axis_index("s")
    wid = sid * 2 + cid
    pltpu.sync_copy(dinv_hbm, dv)

    _zero_rows(zrows, C, H)
    row0 = sid * RPS
    _zero_acc(zrows, acc_sh, row0)
    plsc.subcore_barrier()

    zi = jnp.zeros((16,), jnp.int32)
    ebase = wid * EPT

    def edge_pass(h_hbm):
        @pl.loop(0, CPT)
        def _(c):
            off = ebase + c * C
            pltpu.sync_copy(src_hbm.at[pl.ds(off, C)], sidx)
            pltpu.sync_copy(dst_hbm.at[pl.ds(off, C)], didx)

            @pl.loop(0, 8)
            def _(g):
                s16 = sidx[pl.ds(g * 16, 16)]
                d16 = didx[pl.ds(g * 16, 16)]
                n16 = plsc.load_gather(dv, [s16]) * plsc.load_gather(dv, [d16])
                nbuf[pl.ds(g * 16, 16)] = n16

            pltpu.sync_copy(h_hbm.at[sidx], rows)

            @pl.loop(0, 8)
            def _(g):
                for j in range(16):
                    nj = plsc.load_gather(nbuf, [zi + (g * 16 + j)])
                    for r in range(H // 16):
                        rows[g * 16 + j, pl.ds(r * 16, 16)] = (
                            rows[g * 16 + j, pl.ds(r * 16, 16)] * nj
                        )

            pltpu.sync_copy(rows, acc_sh.at[didx], add=True)

    def write_acc(half):
        @pl.loop(0, RPS // C)
        def _(b):
            r0 = row0 + b * C
            pltpu.sync_copy(acc_sh.at[pl.ds(r0, C)],
                            acc_out.at[cid, half, pl.ds(r0, C)])

    edge_pass(h2a_hbm)
    plsc.subcore_barrier()
    write_acc(0)
    plsc.subcore_barrier()
    _zero_acc(zrows, acc_sh, row0)
    plsc.subcore_barrier()
    edge_pass(h2b_hbm)
    plsc.subcore_barrier()
    write_acc(1)


def _sc_gcn_call(h2a, h2b, src_p, dst_p, dinv):
    kfn = pl.kernel(
        _sc_gcn_body,
        out_type=jax.ShapeDtypeStruct((2, 2, N_PAD, H), jnp.float32),
        mesh=_sc_mesh(),
        scratch_types=[
            pltpu.VMEM((N_PAD,), jnp.float32),
            pltpu.VMEM((C,), jnp.int32),
            pltpu.VMEM((C,), jnp.int32),
            pltpu.VMEM((C, H), jnp.float32),
            pltpu.VMEM((C, H), jnp.float32),
            pltpu.VMEM((C,), jnp.float32),
            pltpu.VMEM_SHARED((N_PAD, H), jnp.float32),
        ],
        compiler_params=_SC_PARAMS,
    )
    return kfn(h2a, h2b, src_p, dst_p, dinv)


def kernel(x, edge_index, batch, W1, att_src, att_dst, b1, W2, b2, Wfc, bfc):
    x_pad = jnp.pad(x, ((0, N_PAD - N), (0, 0)))
    pad_idx = (N + (jnp.arange(E_PAD - E, dtype=jnp.int32) % 16)).astype(jnp.int32)
    src_p = jnp.concatenate([edge_index[0], pad_idx])
    dst_p = jnp.concatenate([edge_index[1], pad_idx])
    batch3 = jnp.concatenate(
        [batch, jnp.full((N_PAD - N,), G, dtype=batch.dtype)]
    ).reshape(NBLK, 1, BLK)

    h0, h1, a_src, a_dst = _tc1_call(x_pad, W1, att_src, att_dst)
    acc1, scal = _sc_gat_call(h0, h1, src_p, dst_p, a_src, a_dst)
    h2a, h2b, dinv = _tc2_call(acc1, scal, b1, W2)
    acc2 = _sc_gcn_call(h2a, h2b, src_p, dst_p, dinv)
    sums, cnts = _tc3a_call(acc2, h2a, h2b, dinv, b2, batch3)
    return _tc3b_call(sums, cnts, Wfc, bfc)


# speedup vs baseline: 16.1993x; 1.6537x over previous
import jax
import jax.numpy as jnp
from jax.experimental import pallas as pl
from jax.experimental.pallas import tpu as pltpu
from jax.experimental.pallas import tpu_sc as plsc

N = 10000
D = 128
H = 64
G = 64
N_PAD = 10240
E = 320000
NT = 32
C = 128
K = 4
CPT = 80
ITERS = CPT // K
EPT = C * CPT
E_PAD = NT * EPT
BLK = 256
NBLK = N_PAD // BLK
RPS = N_PAD // 16

_HI = jax.lax.Precision.HIGHEST


def _tc1_body(x_ref, w1_ref, asrc_ref, adst_ref, h0_ref, h1_ref, as_ref, ad_ref):
    h = jnp.dot(x_ref[...], w1_ref[...], precision=_HI)
    h0_ref[...] = h[:, :H]
    h1_ref[...] = h[:, H:]
    as_ref[...] = jnp.dot(h, asrc_ref[...], precision=_HI)
    ad_ref[...] = jnp.dot(h, adst_ref[...], precision=_HI)


def _tc1_call(x_pad, W1, att_src, att_dst):
    return pl.pallas_call(
        _tc1_body,
        grid=(NBLK,),
        in_specs=[
            pl.BlockSpec((BLK, D), lambda i: (i, 0)),
            pl.BlockSpec((D, D), lambda i: (0, 0)),
            pl.BlockSpec((D,), lambda i: (0,)),
            pl.BlockSpec((D,), lambda i: (0,)),
        ],
        out_specs=[
            pl.BlockSpec((BLK, H), lambda i: (i, 0)),
            pl.BlockSpec((BLK, H), lambda i: (i, 0)),
            pl.BlockSpec((BLK,), lambda i: (i,)),
            pl.BlockSpec((BLK,), lambda i: (i,)),
        ],
        out_shape=[
            jax.ShapeDtypeStruct((N_PAD, H), jnp.float32),
            jax.ShapeDtypeStruct((N_PAD, H), jnp.float32),
            jax.ShapeDtypeStruct((N_PAD,), jnp.float32),
            jax.ShapeDtypeStruct((N_PAD,), jnp.float32),
        ],
    )(x_pad, W1, att_src, att_dst)


def _tc2_body(acc_ref, scal_ref, b1_ref, w2_ref, h2a_ref, h2b_ref, dinv_ref):
    a = jnp.concatenate(
        [acc_ref[0, 0] + acc_ref[1, 0], acc_ref[0, 1] + acc_ref[1, 1]], axis=-1
    )
    den = scal_ref[0, :, 0] + scal_ref[1, :, 0]
    deg = scal_ref[0, :, 1] + scal_ref[1, :, 1] + 1.0
    x2 = jnp.maximum(a / (den + 1e-16)[:, None] + b1_ref[...], 0.0)
    h2 = jnp.dot(x2, w2_ref[...], precision=_HI)
    h2a_ref[...] = h2[:, :H]
    h2b_ref[...] = h2[:, H:]
    dinv_ref[...] = 1.0 / jnp.sqrt(deg)


def _tc2_call(acc1, scal, b1, W2):
    return pl.pallas_call(
        _tc2_body,
        grid=(NBLK,),
        in_specs=[
            pl.BlockSpec((2, 2, BLK, H), lambda i: (0, 0, i, 0)),
            pl.BlockSpec((2, BLK, 16), lambda i: (0, i, 0)),
            pl.BlockSpec((D,), lambda i: (0,)),
            pl.BlockSpec((D, D), lambda i: (0, 0)),
        ],
        out_specs=[
            pl.BlockSpec((BLK, H), lambda i: (i, 0)),
            pl.BlockSpec((BLK, H), lambda i: (i, 0)),
            pl.BlockSpec((BLK,), lambda i: (i,)),
        ],
        out_shape=[
            jax.ShapeDtypeStruct((N_PAD, H), jnp.float32),
            jax.ShapeDtypeStruct((N_PAD, H), jnp.float32),
            jax.ShapeDtypeStruct((N_PAD,), jnp.float32),
        ],
    )(acc1, scal, b1, W2)


def _tc3a_body(acc2_ref, h2a_ref, h2b_ref, dinv_ref, b2_ref, batch_ref,
               sums_ref, cnts_ref):
    i = pl.program_id(0)
    dinv = dinv_ref[...]
    h2 = jnp.concatenate([h2a_ref[...], h2b_ref[...]], axis=-1)
    a2 = jnp.concatenate(
        [acc2_ref[0, 0] + acc2_ref[1, 0], acc2_ref[0, 1] + acc2_ref[1, 1]], axis=-1
    )
    x3 = jnp.maximum(a2 + (dinv * dinv)[:, None] * h2 + b2_ref[...], 0.0)
    bt = batch_ref[0, 0, :]
    oh = (bt[:, None] == jax.lax.broadcasted_iota(jnp.int32, (BLK, G), 1)).astype(jnp.float32)
    ps = jax.lax.dot_general(oh, x3, (((0,), (0,)), ((), ())), precision=_HI)
    cnt = jnp.sum(oh, axis=0)

    @pl.when(i == 0)
    def _():
        sums_ref[...] = jnp.zeros_like(sums_ref)
        cnts_ref[...] = jnp.zeros_like(cnts_ref)

    sums_ref[...] += ps
    cnts_ref[...] += jnp.broadcast_to(cnt[:, None], (G, D))


def _tc3a_call(acc2, h2a, h2b, dinv, b2, batch3):
    return pl.pallas_call(
        _tc3a_body,
        grid=(NBLK,),
        in_specs=[
            pl.BlockSpec((2, 2, BLK, H), lambda i: (0, 0, i, 0)),
            pl.BlockSpec((BLK, H), lambda i: (i, 0)),
            pl.BlockSpec((BLK, H), lambda i: (i, 0)),
            pl.BlockSpec((BLK,), lambda i: (i,)),
            pl.BlockSpec((D,), lambda i: (0,)),
            pl.BlockSpec((1, 1, BLK), lambda i: (i, 0, 0)),
        ],
        out_specs=[
            pl.BlockSpec((G, D), lambda i: (0, 0)),
            pl.BlockSpec((G, D), lambda i: (0, 0)),
        ],
        out_shape=[
            jax.ShapeDtypeStruct((G, D), jnp.float32),
            jax.ShapeDtypeStruct((G, D), jnp.float32),
        ],
    )(acc2, h2a, h2b, dinv, b2, batch3)


def _tc3b_body(sums_ref, cnts_ref, wfc_ref, bfc_ref, out_ref):
    g = sums_ref[...] / jnp.maximum(cnts_ref[...], 1.0)
    out_ref[...] = jnp.dot(g, wfc_ref[...], precision=_HI) + bfc_ref[...]


def _tc3b_call(sums, cnts, Wfc, bfc):
    return pl.pallas_call(
        _tc3b_body,
        out_shape=jax.ShapeDtypeStruct((G, Wfc.shape[1]), jnp.float32),
    )(sums, cnts, Wfc, bfc)


def _sc_mesh():
    return plsc.VectorSubcoreMesh(core_axis_name="c", subcore_axis_name="s")


_SC_PARAMS = pltpu.CompilerParams(
    needs_layout_passes=False, use_tc_tiling_on_sc=False
)


def _zero_rows(ref, nrows, width):
    zf = jnp.zeros((16,), jnp.float32)

    @pl.loop(0, nrows)
    def _(r):
        for k in range(width // 16):
            ref[r, pl.ds(k * 16, 16)] = zf


def _zero_acc(zrows, acc_sh, row0):
    @pl.loop(0, RPS // C)
    def _(b):
        pltpu.sync_copy(zrows, acc_sh.at[pl.ds(row0 + b * C, C)])


def _sc_gat_body(h0_hbm, h1_hbm, src_hbm, dst_hbm, as_hbm, ad_hbm,
                 acc_out, scal_out,
                 asv, adv, sidx, rows3, zrows, pbuf, scalbuf3, z16,
                 acc_sh, scal_sh, isem, gsem, ssem, *didx):
    cid = jax.lax.axis_index("c")
    sid = jax.lax.axis_index("s")
    wid = sid * 2 + cid
    pltpu.sync_copy(as_hbm, asv)
    pltpu.sync_copy(ad_hbm, adv)
    zf = jnp.zeros((16,), jnp.float32)
    lane1 = (jnp.arange(16, dtype=jnp.int32) == 1).astype(jnp.float32)

    _zero_rows(zrows, C, H)

    @pl.loop(0, 64)
    def _(r):
        z16[r] = zf

    for k in range(K):
        @pl.loop(0, C)
        def _(r, k=k):
            scalbuf3[k, r] = lane1

    row0 = sid * RPS
    _zero_acc(zrows, acc_sh, row0)

    @pl.loop(0, 10)
    def _(b):
        pltpu.sync_copy(z16, scal_sh.at[pl.ds(row0 + b * 64, 64)])

    plsc.subcore_barrier()

    iota16 = jnp.arange(16, dtype=jnp.int32)
    zi = jnp.zeros((16,), jnp.int32)

    def edge_pass(h_hbm, do_scal):
        @pl.loop(0, ITERS)
        def _(it):
            ebase = wid * EPT + it * (K * C)
            ci = pltpu.async_copy(src_hbm.at[pl.ds(ebase, K * C)], sidx, isem)
            cds = [
                pltpu.async_copy(
                    dst_hbm.at[pl.ds(ebase + k * C, C)], didx[k], isem
                )
                for k in range(K)
            ]
            ci.wait()
            for cd in cds:
                cd.wait()
            gathers = [
                pltpu.async_copy(
                    h_hbm.at[sidx.at[pl.ds(k * C, C)]], rows3.at[k], gsem.at[k]
                )
                for k in range(K)
            ]
            for k in range(K):
                @pl.loop(0, 8)
                def _(g, k=k):
                    s16 = sidx[pl.ds(k * C + g * 16, 16)]
                    d16 = didx[k][pl.ds(g * 16, 16)]
                    e16 = plsc.load_gather(asv, [s16]) + plsc.load_gather(adv, [d16])
                    e16 = jnp.where(e16 < 0, e16 * 0.2, e16)
                    p16 = jnp.exp(e16)
                    pbuf[pl.ds(k * C + g * 16, 16)] = p16
                    if do_scal:
                        plsc.store_scatter(
                            scalbuf3, [zi + k, g * 16 + iota16, zi], p16
                        )

            scats = []
            for k in range(K):
                gathers[k].wait()

                @pl.loop(0, 8)
                def _(g, k=k):
                    for j in range(16):
                        pj = plsc.load_gather(pbuf, [zi + (k * C + g * 16 + j)])
                        for r in range(H // 16):
                            rows3[k, g * 16 + j, pl.ds(r * 16, 16)] = (
                                rows3[k, g * 16 + j, pl.ds(r * 16, 16)] * pj
                            )

                scats.append(pltpu.async_copy(
                    rows3.at[k], acc_sh.at[didx[k]], ssem, add=True))
                if do_scal:
                    scats.append(pltpu.async_copy(
                        scalbuf3.at[k], scal_sh.at[didx[k]], ssem, add=True))
            for cp in scats:
                cp.wait()

    def write_acc(half):
        @pl.loop(0, RPS // C)
        def _(b):
            r0 = row0 + b * C
            pltpu.sync_copy(acc_sh.at[pl.ds(r0, C)],
                            acc_out.at[cid, half, pl.ds(r0, C)])

    edge_pass(h0_hbm, do_scal=True)
    plsc.subcore_barrier()
    write_acc(0)

    @pl.loop(0, 10)
    def _(b):
        r0 = row0 + b * 64
        pltpu.sync_copy(scal_sh.at[pl.ds(r0, 64)], scal_out.at[cid, pl.ds(r0, 64)])

    plsc.subcore_barrier()
    _zero_acc(zrows, acc_sh, row0)
    plsc.subcore_barrier()
    edge_pass(h1_hbm, do_scal=False)
    plsc.subcore_barrier()
    write_acc(1)


def _sc_gat_call(h0, h1, src_p, dst_p, a_src, a_dst):
    kfn = pl.kernel(
        _sc_gat_body,
        out_type=[
            jax.ShapeDtypeStruct((2, 2, N_PAD, H), jnp.float32),
            jax.ShapeDtypeStruct((2, N_PAD, 16), jnp.float32),
        ],
        mesh=_sc_mesh(),
        scratch_types=[
            pltpu.VMEM((N_PAD,), jnp.float32),
            pltpu.VMEM((N_PAD,), jnp.float32),
            pltpu.VMEM((K * C,), jnp.int32),
            pltpu.VMEM((K, C, H), jnp.float32),
            pltpu.VMEM((C, H), jnp.float32),
            pltpu.VMEM((K * C,), jnp.float32),
            pltpu.VMEM((K, C, 16), jnp.float32),
            pltpu.VMEM((64, 16), jnp.float32),
            pltpu.VMEM_SHARED((N_PAD, H), jnp.float32),
            pltpu.VMEM_SHARED((N_PAD, 16), jnp.float32),
            pltpu.SemaphoreType.DMA,
            pltpu.SemaphoreType.DMA((K,)),
            pltpu.SemaphoreType.DMA,
        ] + [pltpu.VMEM((C,), jnp.int32) for _ in range(K)],
        compiler_params=_SC_PARAMS,
    )
    return kfn(h0, h1, src_p, dst_p, a_src, a_dst)


def _sc_gcn_body(h2a_hbm, h2b_hbm, src_hbm, dst_hbm, dinv_hbm, acc_out,
                 dv, sidx, rows3, zrows, nbuf, acc_sh, isem, gsem, ssem, *didx):
    cid = jax.lax.axis_index("c")
    sid = jax.lax.axis_index("s")
    wid = sid * 2 + cid
    pltpu.sync_copy(dinv_hbm, dv)

    _zero_rows(zrows, C, H)
    row0 = sid * RPS
    _zero_acc(zrows, acc_sh, row0)
    plsc.subcore_barrier()

    zi = jnp.zeros((16,), jnp.int32)

    def edge_pass(h_hbm):
        @pl.loop(0, ITERS)
        def _(it):
            ebase = wid * EPT + it * (K * C)
            ci = pltpu.async_copy(src_hbm.at[pl.ds(ebase, K * C)], sidx, isem)
            cds = [
                pltpu.async_copy(
                    dst_hbm.at[pl.ds(ebase + k * C, C)], didx[k], isem
                )
                for k in range(K)
            ]
            ci.wait()
            for cd in cds:
                cd.wait()
            gathers = [
                pltpu.async_copy(
                    h_hbm.at[sidx.at[pl.ds(k * C, C)]], rows3.at[k], gsem.at[k]
                )
                for k in range(K)
            ]
            for k in range(K):
                @pl.loop(0, 8)
                def _(g, k=k):
                    s16 = sidx[pl.ds(k * C + g * 16, 16)]
                    d16 = didx[k][pl.ds(g * 16, 16)]
                    n16 = plsc.load_gather(dv, [s16]) * plsc.load_gather(dv, [d16])
                    nbuf[pl.ds(k * C + g * 16, 16)] = n16

            scats = []
            for k in range(K):
                gathers[k].wait()

                @pl.loop(0, 8)
                def _(g, k=k):
                    for j in range(16):
                        nj = plsc.load_gather(nbuf, [zi + (k * C + g * 16 + j)])
                        for r in range(H // 16):
                            rows3[k, g * 16 + j, pl.ds(r * 16, 16)] = (
                                rows3[k, g * 16 + j, pl.ds(r * 16, 16)] * nj
                            )

                scats.append(pltpu.async_copy(
                    rows3.at[k], acc_sh.at[didx[k]], ssem, add=True))
            for cp in scats:
                cp.wait()

    def write_acc(half):
        @pl.loop(0, RPS // C)
        def _(b):
            r0 = row0 + b * C
            pltpu.sync_copy(acc_sh.at[pl.ds(r0, C)],
                            acc_out.at[cid, half, pl.ds(r0, C)])

    edge_pass(h2a_hbm)
    plsc.subcore_barrier()
    write_acc(0)
    plsc.subcore_barrier()
    _zero_acc(zrows, acc_sh, row0)
    plsc.subcore_barrier()
    edge_pass(h2b_hbm)
    plsc.subcore_barrier()
    write_acc(1)


def _sc_gcn_call(h2a, h2b, src_p, dst_p, dinv):
    kfn = pl.kernel(
        _sc_gcn_body,
        out_type=jax.ShapeDtypeStruct((2, 2, N_PAD, H), jnp.float32),
        mesh=_sc_mesh(),
        scratch_types=[
            pltpu.VMEM((N_PAD,), jnp.float32),
            pltpu.VMEM((K * C,), jnp.int32),
            pltpu.VMEM((K, C, H), jnp.float32),
            pltpu.VMEM((C, H), jnp.float32),
            pltpu.VMEM((K * C,), jnp.float32),
            pltpu.VMEM_SHARED((N_PAD, H), jnp.float32),
            pltpu.SemaphoreType.DMA,
            pltpu.SemaphoreType.DMA((K,)),
            pltpu.SemaphoreType.DMA,
        ] + [pltpu.VMEM((C,), jnp.int32) for _ in range(K)],
        compiler_params=_SC_PARAMS,
    )
    return kfn(h2a, h2b, src_p, dst_p, dinv)


def kernel(x, edge_index, batch, W1, att_src, att_dst, b1, W2, b2, Wfc, bfc):
    x_pad = jnp.pad(x, ((0, N_PAD - N), (0, 0)))
    pad_idx = (N + (jnp.arange(E_PAD - E, dtype=jnp.int32) % 16)).astype(jnp.int32)
    src_p = jnp.concatenate([edge_index[0], pad_idx])
    dst_p = jnp.concatenate([edge_index[1], pad_idx])
    batch3 = jnp.concatenate(
        [batch, jnp.full((N_PAD - N,), G, dtype=batch.dtype)]
    ).reshape(NBLK, 1, BLK)

    h0, h1, a_src, a_dst = _tc1_call(x_pad, W1, att_src, att_dst)
    acc1, scal = _sc_gat_call(h0, h1, src_p, dst_p, a_src, a_dst)
    h2a, h2b, dinv = _tc2_call(acc1, scal, b1, W2)
    acc2 = _sc_gcn_call(h2a, h2b, src_p, dst_p, dinv)
    sums, cnts = _tc3a_call(acc2, h2a, h2b, dinv, b2, batch3)
    return _tc3b_call(sums, cnts, Wfc, bfc)


# speedup vs baseline: 22.2127x; 1.3712x over previous
import jax
import jax.numpy as jnp
from jax.experimental import pallas as pl
from jax.experimental.pallas import tpu as pltpu
from jax.experimental.pallas import tpu_sc as plsc

N = 10000
D = 128
H = 64
G = 64
N_PAD = 10240
E = 320000
NT = 32
C = 128
K = 2
CPT = 80
ITERS = CPT // K
EPT = C * CPT
E_PAD = NT * EPT
BLK = 256
NBLK = N_PAD // BLK
RPS = N_PAD // 16

_HI = jax.lax.Precision.HIGHEST


def _tc1_body(x_ref, w1_ref, asrc_ref, adst_ref, h0_ref, h1_ref, as_ref, ad_ref):
    h = jnp.dot(x_ref[...], w1_ref[...], precision=_HI)
    h0_ref[...] = h[:, :H]
    h1_ref[...] = h[:, H:]
    as_ref[...] = jnp.dot(h, asrc_ref[...], precision=_HI)
    ad_ref[...] = jnp.dot(h, adst_ref[...], precision=_HI)


def _tc1_call(x_pad, W1, att_src, att_dst):
    return pl.pallas_call(
        _tc1_body,
        grid=(NBLK,),
        in_specs=[
            pl.BlockSpec((BLK, D), lambda i: (i, 0)),
            pl.BlockSpec((D, D), lambda i: (0, 0)),
            pl.BlockSpec((D,), lambda i: (0,)),
            pl.BlockSpec((D,), lambda i: (0,)),
        ],
        out_specs=[
            pl.BlockSpec((BLK, H), lambda i: (i, 0)),
            pl.BlockSpec((BLK, H), lambda i: (i, 0)),
            pl.BlockSpec((BLK,), lambda i: (i,)),
            pl.BlockSpec((BLK,), lambda i: (i,)),
        ],
        out_shape=[
            jax.ShapeDtypeStruct((N_PAD, H), jnp.float32),
            jax.ShapeDtypeStruct((N_PAD, H), jnp.float32),
            jax.ShapeDtypeStruct((N_PAD,), jnp.float32),
            jax.ShapeDtypeStruct((N_PAD,), jnp.float32),
        ],
    )(x_pad, W1, att_src, att_dst)


def _tc2_body(acc_ref, scal_ref, b1_ref, w2_ref, h2a_ref, h2b_ref, dinv_ref):
    a = jnp.concatenate(
        [acc_ref[0, 0] + acc_ref[1, 0], acc_ref[0, 1] + acc_ref[1, 1]], axis=-1
    )
    den = scal_ref[0, :, 0] + scal_ref[1, :, 0]
    deg = scal_ref[0, :, 1] + scal_ref[1, :, 1] + 1.0
    x2 = jnp.maximum(a / (den + 1e-16)[:, None] + b1_ref[...], 0.0)
    h2 = jnp.dot(x2, w2_ref[...], precision=_HI)
    h2a_ref[...] = h2[:, :H]
    h2b_ref[...] = h2[:, H:]
    dinv_ref[...] = 1.0 / jnp.sqrt(deg)


def _tc2_call(acc1, scal, b1, W2):
    return pl.pallas_call(
        _tc2_body,
        grid=(NBLK,),
        in_specs=[
            pl.BlockSpec((2, 2, BLK, H), lambda i: (0, 0, i, 0)),
            pl.BlockSpec((2, BLK, 16), lambda i: (0, i, 0)),
            pl.BlockSpec((D,), lambda i: (0,)),
            pl.BlockSpec((D, D), lambda i: (0, 0)),
        ],
        out_specs=[
            pl.BlockSpec((BLK, H), lambda i: (i, 0)),
            pl.BlockSpec((BLK, H), lambda i: (i, 0)),
            pl.BlockSpec((BLK,), lambda i: (i,)),
        ],
        out_shape=[
            jax.ShapeDtypeStruct((N_PAD, H), jnp.float32),
            jax.ShapeDtypeStruct((N_PAD, H), jnp.float32),
            jax.ShapeDtypeStruct((N_PAD,), jnp.float32),
        ],
    )(acc1, scal, b1, W2)


def _tc3a_body(acc2_ref, h2a_ref, h2b_ref, dinv_ref, b2_ref, batch_ref,
               sums_ref, cnts_ref):
    i = pl.program_id(0)
    dinv = dinv_ref[...]
    h2 = jnp.concatenate([h2a_ref[...], h2b_ref[...]], axis=-1)
    a2 = jnp.concatenate(
        [acc2_ref[0, 0] + acc2_ref[1, 0], acc2_ref[0, 1] + acc2_ref[1, 1]], axis=-1
    )
    x3 = jnp.maximum(a2 + (dinv * dinv)[:, None] * h2 + b2_ref[...], 0.0)
    bt = batch_ref[0, 0, :]
    oh = (bt[:, None] == jax.lax.broadcasted_iota(jnp.int32, (BLK, G), 1)).astype(jnp.float32)
    ps = jax.lax.dot_general(oh, x3, (((0,), (0,)), ((), ())), precision=_HI)
    cnt = jnp.sum(oh, axis=0)

    @pl.when(i == 0)
    def _():
        sums_ref[...] = jnp.zeros_like(sums_ref)
        cnts_ref[...] = jnp.zeros_like(cnts_ref)

    sums_ref[...] += ps
    cnts_ref[...] += jnp.broadcast_to(cnt[:, None], (G, D))


def _tc3a_call(acc2, h2a, h2b, dinv, b2, batch3):
    return pl.pallas_call(
        _tc3a_body,
        grid=(NBLK,),
        in_specs=[
            pl.BlockSpec((2, 2, BLK, H), lambda i: (0, 0, i, 0)),
            pl.BlockSpec((BLK, H), lambda i: (i, 0)),
            pl.BlockSpec((BLK, H), lambda i: (i, 0)),
            pl.BlockSpec((BLK,), lambda i: (i,)),
            pl.BlockSpec((D,), lambda i: (0,)),
            pl.BlockSpec((1, 1, BLK), lambda i: (i, 0, 0)),
        ],
        out_specs=[
            pl.BlockSpec((G, D), lambda i: (0, 0)),
            pl.BlockSpec((G, D), lambda i: (0, 0)),
        ],
        out_shape=[
            jax.ShapeDtypeStruct((G, D), jnp.float32),
            jax.ShapeDtypeStruct((G, D), jnp.float32),
        ],
    )(acc2, h2a, h2b, dinv, b2, batch3)


def _tc3b_body(sums_ref, cnts_ref, wfc_ref, bfc_ref, out_ref):
    g = sums_ref[...] / jnp.maximum(cnts_ref[...], 1.0)
    out_ref[...] = jnp.dot(g, wfc_ref[...], precision=_HI) + bfc_ref[...]


def _tc3b_call(sums, cnts, Wfc, bfc):
    return pl.pallas_call(
        _tc3b_body,
        out_shape=jax.ShapeDtypeStruct((G, Wfc.shape[1]), jnp.float32),
    )(sums, cnts, Wfc, bfc)


def _sc_mesh():
    return plsc.VectorSubcoreMesh(core_axis_name="c", subcore_axis_name="s")


_SC_PARAMS = pltpu.CompilerParams(
    needs_layout_passes=False, use_tc_tiling_on_sc=False
)


def _zero_rows(ref, nrows, width):
    zf = jnp.zeros((16,), jnp.float32)

    @pl.loop(0, nrows)
    def _(r):
        for k in range(width // 16):
            ref[r, pl.ds(k * 16, 16)] = zf


def _zero_acc(zrows, acc_sh, row0):
    @pl.loop(0, RPS // C)
    def _(b):
        pltpu.sync_copy(zrows, acc_sh.at[pl.ds(row0 + b * C, C)])


def _sc_gat_body(h0_hbm, h1_hbm, src_hbm, dst_hbm, as_hbm, ad_hbm,
                 acc_out, scal_out,
                 asv, adv, sidx, rows3, zrows, pbuf, scalbuf3, z16,
                 acc_sh, scal_sh, isem, gsem, ssem, *didx):
    cid = jax.lax.axis_index("c")
    sid = jax.lax.axis_index("s")
    wid = sid * 2 + cid
    pltpu.sync_copy(as_hbm, asv)
    pltpu.sync_copy(ad_hbm, adv)
    zf = jnp.zeros((16,), jnp.float32)
    lane1 = (jnp.arange(16, dtype=jnp.int32) == 1).astype(jnp.float32)

    _zero_rows(zrows, C, H)

    @pl.loop(0, 64)
    def _(r):
        z16[r] = zf

    for k in range(2 * K):
        @pl.loop(0, C)
        def _(r, k=k):
            scalbuf3[k, r] = lane1

    row0 = sid * RPS
    _zero_acc(zrows, acc_sh, row0)

    @pl.loop(0, 10)
    def _(b):
        pltpu.sync_copy(z16, scal_sh.at[pl.ds(row0 + b * 64, 64)])

    plsc.subcore_barrier()

    iota16 = jnp.arange(16, dtype=jnp.int32)
    zi = jnp.zeros((16,), jnp.int32)

    def edge_pass(h_hbm, do_scal):
        def f_stage(s, it):
            ebase = wid * EPT + it * (K * C)
            o = s * K * C
            ci = pltpu.async_copy(
                src_hbm.at[pl.ds(ebase, K * C)], sidx.at[pl.ds(o, K * C)], isem)
            cds = [
                pltpu.async_copy(
                    dst_hbm.at[pl.ds(ebase + k * C, C)], didx[s * K + k], isem
                )
                for k in range(K)
            ]
            ci.wait()
            for cd in cds:
                cd.wait()
            for k in range(K):
                pltpu.async_copy(
                    h_hbm.at[sidx.at[pl.ds(o + k * C, C)]],
                    rows3.at[s * K + k], gsem.at[s * K + k])

            for k in range(K):
                @pl.loop(0, 8)
                def _(g, k=k):
                    s16 = sidx[pl.ds(o + k * C + g * 16, 16)]
                    d16 = didx[s * K + k][pl.ds(g * 16, 16)]
                    e16 = plsc.load_gather(asv, [s16]) + plsc.load_gather(adv, [d16])
                    e16 = jnp.where(e16 < 0, e16 * 0.2, e16)
                    p16 = jnp.exp(e16)
                    pbuf[pl.ds(o + k * C + g * 16, 16)] = p16
                    if do_scal:
                        plsc.store_scatter(
                            scalbuf3, [zi + (s * K + k), g * 16 + iota16, zi], p16
                        )

        def be_stage(s):
            o = s * K * C
            scats = []
            for k in range(K):
                pltpu.make_async_copy(
                    h_hbm.at[sidx.at[pl.ds(o + k * C, C)]],
                    rows3.at[s * K + k], gsem.at[s * K + k]).wait()

                @pl.loop(0, C)
                def _(e, k=k):
                    pj = plsc.load_gather(pbuf, [zi + (o + k * C + e)])
                    for r in range(H // 16):
                        rows3[s * K + k, e, pl.ds(r * 16, 16)] = (
                            rows3[s * K + k, e, pl.ds(r * 16, 16)] * pj
                        )

                scats.append(pltpu.async_copy(
                    rows3.at[s * K + k], acc_sh.at[didx[s * K + k]],
                    ssem.at[s], add=True))
                if do_scal:
                    scats.append(pltpu.async_copy(
                        scalbuf3.at[s * K + k], scal_sh.at[didx[s * K + k]],
                        ssem.at[s], add=True))
            for cp in scats:
                cp.wait()

        f_stage(0, 0)
        f_stage(1, 1)
        be_stage(0)

        @pl.loop(1, ITERS // 2)
        def _(p):
            f_stage(0, 2 * p)
            be_stage(1)
            f_stage(1, 2 * p + 1)
            be_stage(0)

        be_stage(1)

    def write_acc(half):
        @pl.loop(0, RPS // C)
        def _(b):
            r0 = row0 + b * C
            pltpu.sync_copy(acc_sh.at[pl.ds(r0, C)],
                            acc_out.at[cid, half, pl.ds(r0, C)])

    edge_pass(h0_hbm, do_scal=True)
    plsc.subcore_barrier()
    write_acc(0)

    @pl.loop(0, 10)
    def _(b):
        r0 = row0 + b * 64
        pltpu.sync_copy(scal_sh.at[pl.ds(r0, 64)], scal_out.at[cid, pl.ds(r0, 64)])

    plsc.subcore_barrier()
    _zero_acc(zrows, acc_sh, row0)
    plsc.subcore_barrier()
    edge_pass(h1_hbm, do_scal=False)
    plsc.subcore_barrier()
    write_acc(1)


def _sc_gat_call(h0, h1, src_p, dst_p, a_src, a_dst):
    kfn = pl.kernel(
        _sc_gat_body,
        out_type=[
            jax.ShapeDtypeStruct((2, 2, N_PAD, H), jnp.float32),
            jax.ShapeDtypeStruct((2, N_PAD, 16), jnp.float32),
        ],
        mesh=_sc_mesh(),
        scratch_types=[
            pltpu.VMEM((N_PAD,), jnp.float32),
            pltpu.VMEM((N_PAD,), jnp.float32),
            pltpu.VMEM((2 * K * C,), jnp.int32),
            pltpu.VMEM((2 * K, C, H), jnp.float32),
            pltpu.VMEM((C, H), jnp.float32),
            pltpu.VMEM((2 * K * C,), jnp.float32),
            pltpu.VMEM((2 * K, C, 16), jnp.float32),
            pltpu.VMEM((64, 16), jnp.float32),
            pltpu.VMEM_SHARED((N_PAD, H), jnp.float32),
            pltpu.VMEM_SHARED((N_PAD, 16), jnp.float32),
            pltpu.SemaphoreType.DMA,
            pltpu.SemaphoreType.DMA((2 * K,)),
            pltpu.SemaphoreType.DMA((2,)),
        ] + [pltpu.VMEM((C,), jnp.int32) for _ in range(2 * K)],
        compiler_params=_SC_PARAMS,
    )
    return kfn(h0, h1, src_p, dst_p, a_src, a_dst)


def _sc_gcn_body(h2a_hbm, h2b_hbm, src_hbm, dst_hbm, dinv_hbm, acc_out,
                 dv, sidx, rows3, zrows, nbuf, acc_sh, isem, gsem, ssem, *didx):
    cid = jax.lax.axis_index("c")
    sid = jax.lax.axis_index("s")
    wid = sid * 2 + cid
    pltpu.sync_copy(dinv_hbm, dv)

    _zero_rows(zrows, C, H)
    row0 = sid * RPS
    _zero_acc(zrows, acc_sh, row0)
    plsc.subcore_barrier()

    zi = jnp.zeros((16,), jnp.int32)

    def edge_pass(h_hbm):
        def f_stage(s, it):
            ebase = wid * EPT + it * (K * C)
            o = s * K * C
            ci = pltpu.async_copy(
                src_hbm.at[pl.ds(ebase, K * C)], sidx.at[pl.ds(o, K * C)], isem)
            cds = [
                pltpu.async_copy(
                    dst_hbm.at[pl.ds(ebase + k * C, C)], didx[s * K + k], isem
                )
                for k in range(K)
            ]
            ci.wait()
            for cd in cds:
                cd.wait()
            for k in range(K):
                pltpu.async_copy(
                    h_hbm.at[sidx.at[pl.ds(o + k * C, C)]],
                    rows3.at[s * K + k], gsem.at[s * K + k])

            for k in range(K):
                @pl.loop(0, 8)
                def _(g, k=k):
                    s16 = sidx[pl.ds(o + k * C + g * 16, 16)]
                    d16 = didx[s * K + k][pl.ds(g * 16, 16)]
                    n16 = plsc.load_gather(dv, [s16]) * plsc.load_gather(dv, [d16])
                    nbuf[pl.ds(o + k * C + g * 16, 16)] = n16

        def be_stage(s):
            o = s * K * C
            scats = []
            for k in range(K):
                pltpu.make_async_copy(
                    h_hbm.at[sidx.at[pl.ds(o + k * C, C)]],
                    rows3.at[s * K + k], gsem.at[s * K + k]).wait()

                @pl.loop(0, C)
                def _(e, k=k):
                    nj = plsc.load_gather(nbuf, [zi + (o + k * C + e)])
                    for r in range(H // 16):
                        rows3[s * K + k, e, pl.ds(r * 16, 16)] = (
                            rows3[s * K + k, e, pl.ds(r * 16, 16)] * nj
                        )

                scats.append(pltpu.async_copy(
                    rows3.at[s * K + k], acc_sh.at[didx[s * K + k]],
                    ssem.at[s], add=True))
            for cp in scats:
                cp.wait()

        f_stage(0, 0)
        f_stage(1, 1)
        be_stage(0)

        @pl.loop(1, ITERS // 2)
        def _(p):
            f_stage(0, 2 * p)
            be_stage(1)
            f_stage(1, 2 * p + 1)
            be_stage(0)

        be_stage(1)

    def write_acc(half):
        @pl.loop(0, RPS // C)
        def _(b):
            r0 = row0 + b * C
            pltpu.sync_copy(acc_sh.at[pl.ds(r0, C)],
                            acc_out.at[cid, half, pl.ds(r0, C)])

    edge_pass(h2a_hbm)
    plsc.subcore_barrier()
    write_acc(0)
    plsc.subcore_barrier()
    _zero_acc(zrows, acc_sh, row0)
    plsc.subcore_barrier()
    edge_pass(h2b_hbm)
    plsc.subcore_barrier()
    write_acc(1)


def _sc_gcn_call(h2a, h2b, src_p, dst_p, dinv):
    kfn = pl.kernel(
        _sc_gcn_body,
        out_type=jax.ShapeDtypeStruct((2, 2, N_PAD, H), jnp.float32),
        mesh=_sc_mesh(),
        scratch_types=[
            pltpu.VMEM((N_PAD,), jnp.float32),
            pltpu.VMEM((2 * K * C,), jnp.int32),
            pltpu.VMEM((2 * K, C, H), jnp.float32),
            pltpu.VMEM((C, H), jnp.float32),
            pltpu.VMEM((2 * K * C,), jnp.float32),
            pltpu.VMEM_SHARED((N_PAD, H), jnp.float32),
            pltpu.SemaphoreType.DMA,
            pltpu.SemaphoreType.DMA((2 * K,)),
            pltpu.SemaphoreType.DMA((2,)),
        ] + [pltpu.VMEM((C,), jnp.int32) for _ in range(2 * K)],
        compiler_params=_SC_PARAMS,
    )
    return kfn(h2a, h2b, src_p, dst_p, dinv)


def kernel(x, edge_index, batch, W1, att_src, att_dst, b1, W2, b2, Wfc, bfc):
    x_pad = jnp.pad(x, ((0, N_PAD - N), (0, 0)))
    pad_idx = (N + (jnp.arange(E_PAD - E, dtype=jnp.int32) % 16)).astype(jnp.int32)
    src_p = jnp.concatenate([edge_index[0], pad_idx])
    dst_p = jnp.concatenate([edge_index[1], pad_idx])
    batch3 = jnp.concatenate(
        [batch, jnp.full((N_PAD - N,), G, dtype=batch.dtype)]
    ).reshape(NBLK, 1, BLK)

    h0, h1, a_src, a_dst = _tc1_call(x_pad, W1, att_src, att_dst)
    acc1, scal = _sc_gat_call(h0, h1, src_p, dst_p, a_src, a_dst)
    h2a, h2b, dinv = _tc2_call(acc1, scal, b1, W2)
    acc2 = _sc_gcn_call(h2a, h2b, src_p, dst_p, dinv)
    sums, cnts = _tc3a_call(acc2, h2a, h2b, dinv, b2, batch3)
    return _tc3b_call(sums, cnts, Wfc, bfc)
